# baseline (speedup 1.0000x reference)
import functools

import jax
import jax.numpy as jnp
from jax import lax
from jax.experimental import pallas as pl
from jax.experimental.pallas import tpu as pltpu
from jax.experimental.pallas import tpu_sc as plsc

EMB = 16
NUM_WORKERS = 32
BLOCK = 1024


def _mf_body(nblk, uidx_hbm, iidx_hbm, users_hbm, items_hbm, out_hbm,
             idx_u, idx_i, rows_u, rows_i, out_blk, sem):
    wid = lax.axis_index("s") * 2 + lax.axis_index("c")
    base = wid * (nblk * BLOCK)
    for blk in range(nblk):
        off = base + blk * BLOCK
        pltpu.sync_copy(uidx_hbm.at[pl.ds(off, BLOCK)], idx_u)
        pltpu.sync_copy(iidx_hbm.at[pl.ds(off, BLOCK)], idx_i)
        cu = pltpu.async_copy(users_hbm.at[idx_u], rows_u, sem)
        ci = pltpu.async_copy(items_hbm.at[idx_i], rows_i, sem)
        cu.wait()
        ci.wait()

        lane = lax.iota(jnp.int32, 16)

        def group(g, carry):
            rid = g * 16 + lane
            acc = jnp.zeros((16,), jnp.float32)
            for k in range(EMB):
                ck = jnp.full((16,), k, jnp.int32)
                uk = plsc.load_gather(rows_u, [rid, ck])
                ik = plsc.load_gather(rows_i, [rid, ck])
                acc = acc + uk * ik
            out_blk[pl.ds(g * 16, 16)] = acc
            return carry

        lax.fori_loop(0, BLOCK // 16, group, 0)
        pltpu.sync_copy(out_blk, out_hbm.at[pl.ds(off, BLOCK)])


def kernel(user, item, users, items):
    B, L = user.shape
    n = B * L
    per_w = n // NUM_WORKERS
    nblk = per_w // BLOCK
    assert per_w * NUM_WORKERS == n and nblk * BLOCK == per_w

    mesh = plsc.VectorSubcoreMesh(core_axis_name="c", subcore_axis_name="s")
    f = pl.kernel(
        functools.partial(_mf_body, nblk),
        out_type=jax.ShapeDtypeStruct((n,), jnp.float32),
        mesh=mesh,
        scratch_types=[
            pltpu.VMEM((BLOCK,), jnp.int32),
            pltpu.VMEM((BLOCK,), jnp.int32),
            pltpu.VMEM((BLOCK, EMB), jnp.float32),
            pltpu.VMEM((BLOCK, EMB), jnp.float32),
            pltpu.VMEM((BLOCK,), jnp.float32),
            pltpu.SemaphoreType.DMA,
        ],
        compiler_params=pltpu.CompilerParams(
            use_tc_tiling_on_sc=False, needs_layout_passes=False),
    )
    out = f(user.reshape(-1).astype(jnp.int32),
            item.reshape(-1).astype(jnp.int32),
            users, items)
    return out.reshape(B, L)


# speedup vs baseline: 1.0419x; 1.0419x over previous
import functools

import jax
import jax.numpy as jnp
from jax import lax
from jax.experimental import pallas as pl
from jax.experimental.pallas import tpu as pltpu
from jax.experimental.pallas import tpu_sc as plsc

EMB = 16
NUM_WORKERS = 32
BLOCK = 1024


def _mf_body(nblk, uidx_hbm, iidx_hbm, users_hbm, items_hbm, out_hbm,
             idx_u, idx_i, rows_u, rows_i, out_blk,
             sem_iu, sem_ii, sem_u, sem_i, sem_o):
    wid = lax.axis_index("s") * 2 + lax.axis_index("c")
    base = wid * (nblk * BLOCK)

    def fire_idx(blk):
        p = blk % 2
        off = base + blk * BLOCK
        cu = pltpu.async_copy(uidx_hbm.at[pl.ds(off, BLOCK)], idx_u.at[p],
                              sem_iu)
        ci = pltpu.async_copy(iidx_hbm.at[pl.ds(off, BLOCK)], idx_i.at[p],
                              sem_ii)
        return cu, ci

    def fire_rows(blk):
        p = blk % 2
        cu = pltpu.async_copy(users_hbm.at[idx_u.at[p]], rows_u.at[p], sem_u)
        ci = pltpu.async_copy(items_hbm.at[idx_i.at[p]], rows_i.at[p], sem_i)
        return cu, ci

    lane = lax.iota(jnp.int32, 16)
    cols = [jnp.full((16,), k, jnp.int32) for k in range(EMB)]

    i0 = fire_idx(0)
    i1 = fire_idx(1)
    i0[0].wait()
    i0[1].wait()
    g_pend = fire_rows(0)
    o_pend = None
    idx_pend = i1

    for blk in range(nblk):
        p = blk % 2
        g_pend[0].wait()
        g_pend[1].wait()
        if blk + 1 < nblk:
            idx_pend[0].wait()
            idx_pend[1].wait()
            if blk + 2 < nblk:
                idx_pend = fire_idx(blk + 2)
            g_pend = fire_rows(blk + 1)
        if o_pend is not None:
            o_pend.wait()

        ru = rows_u.at[p]
        ri = rows_i.at[p]

        def group(g, carry, _ru=ru, _ri=ri, _p=p):
            rid = g * 16 + lane
            acc = None
            for k in range(EMB):
                uk = plsc.load_gather(_ru, [rid, cols[k]])
                ik = plsc.load_gather(_ri, [rid, cols[k]])
                prod = uk * ik
                acc = prod if acc is None else acc + prod
            out_blk[_p, pl.ds(g * 16, 16)] = acc
            return carry

        lax.fori_loop(0, BLOCK // 16, group, 0, unroll=2)

        off = base + blk * BLOCK
        o_pend = pltpu.async_copy(out_blk.at[p],
                                  out_hbm.at[pl.ds(off, BLOCK)], sem_o)
    o_pend.wait()


def kernel(user, item, users, items):
    B, L = user.shape
    n = B * L
    per_w = n // NUM_WORKERS
    nblk = per_w // BLOCK
    assert per_w * NUM_WORKERS == n and nblk * BLOCK == per_w

    mesh = plsc.VectorSubcoreMesh(core_axis_name="c", subcore_axis_name="s")
    f = pl.kernel(
        functools.partial(_mf_body, nblk),
        out_type=jax.ShapeDtypeStruct((n,), jnp.float32),
        mesh=mesh,
        scratch_types=[
            pltpu.VMEM((2, BLOCK), jnp.int32),
            pltpu.VMEM((2, BLOCK), jnp.int32),
            pltpu.VMEM((2, BLOCK, EMB), jnp.float32),
            pltpu.VMEM((2, BLOCK, EMB), jnp.float32),
            pltpu.VMEM((2, BLOCK), jnp.float32),
            pltpu.SemaphoreType.DMA,
            pltpu.SemaphoreType.DMA,
            pltpu.SemaphoreType.DMA,
            pltpu.SemaphoreType.DMA,
            pltpu.SemaphoreType.DMA,
        ],
        compiler_params=pltpu.CompilerParams(
            use_tc_tiling_on_sc=False, needs_layout_passes=False),
    )
    out = f(user.reshape(-1).astype(jnp.int32),
            item.reshape(-1).astype(jnp.int32),
            users, items)
    return out.reshape(B, L)


# speedup vs baseline: 1.4218x; 1.3646x over previous
import functools

import jax
import jax.numpy as jnp
from jax import lax
from jax.experimental import pallas as pl
from jax.experimental.pallas import tpu as pltpu
from jax.experimental.pallas import tpu_sc as plsc

EMB = 16
NUM_WORKERS = 32
BLOCK = 1024
NCB = 15625
CCB = 5
CP = CCB * 128


def _mf_body(nblk, nrows, uidx_hbm, iidx_hbm, x_hbm, out_hbm,
             scr, tin, rm, idx_u, idx_i, rows_u, rows_i, out_blk,
             sem_t, sem_r, sem_iu, sem_ii, sem_u, sem_i, sem_o):
    cid = lax.axis_index("c")
    sid = lax.axis_index("s")
    wid = sid * 2 + cid
    lane = lax.iota(jnp.int32, 16)
    cols = [jnp.full((16,), k, jnp.int32) for k in range(EMB)]

    nch_all = NCB // CCB
    base_ch = nch_all // 16
    rem_ch = nch_all - base_ch * 16
    nch = jnp.where(sid < rem_ch, base_ch + 1, base_ch)
    ch0 = sid * base_ch + jnp.minimum(sid, rem_ch)

    def fire_in(ch, p):
        cb = (ch0 + ch) * CCB
        return (pltpu.async_copy(x_hbm.at[0, pl.ds(cb, CCB)], tin.at[p, 0],
                                 sem_t),
                pltpu.async_copy(x_hbm.at[1, pl.ds(cb, CCB)], tin.at[p, 1],
                                 sem_t))

    def chunk_body(ch, carry):
        p = ch % 2
        pltpu.make_async_copy(x_hbm.at[0, pl.ds(0, CCB)], tin.at[p, 0],
                              sem_t).wait()
        pltpu.make_async_copy(x_hbm.at[1, pl.ds(0, CCB)], tin.at[p, 1],
                              sem_t).wait()

        @pl.when(ch >= 2)
        def _():
            pltpu.make_async_copy(rm.at[p], scr.at[0, pl.ds(0, CP)],
                                  sem_r).wait()

        def group(g, _):
            cbl = g // 8
            cc = (g % 8) * 16
            rowv = g * 16 + lane
            for k in range(EMB):
                v = tin[p, k // 8, cbl, k % 8, pl.ds(cc, 16)]
                plsc.store_scatter(rm.at[p], [rowv, cols[k]], v)
            return _

        lax.fori_loop(0, CP // 16, group, 0, unroll=2)

        @pl.when(ch + 2 < nch)
        def _():
            fire_in(ch + 2, p)

        p0 = (ch0 + ch) * CP
        pltpu.async_copy(rm.at[p], scr.at[cid, pl.ds(p0, CP)], sem_r)
        return carry

    fire_in(0, 0)

    @pl.when(nch > 1)
    def _():
        fire_in(1, 1)

    lax.fori_loop(0, nch, chunk_body, 0)
    pltpu.make_async_copy(rm.at[0], scr.at[0, pl.ds(0, CP)], sem_r).wait()

    @pl.when(nch > 1)
    def _():
        pltpu.make_async_copy(rm.at[1], scr.at[0, pl.ds(0, CP)], sem_r).wait()

    plsc.subcore_barrier()

    my_scr = scr.at[cid]
    base = wid * (nblk * BLOCK)

    def fire_idx(blk):
        p = blk % 2
        off = base + blk * BLOCK
        cu = pltpu.async_copy(uidx_hbm.at[pl.ds(off, BLOCK)], idx_u.at[p],
                              sem_iu)
        ci = pltpu.async_copy(iidx_hbm.at[pl.ds(off, BLOCK)], idx_i.at[p],
                              sem_ii)
        return cu, ci

    def adjust_items(p):
        def add_off(g, _):
            s = g * 16
            idx_i[p, pl.ds(s, 16)] = idx_i[p, pl.ds(s, 16)] + nrows
            return _
        lax.fori_loop(0, BLOCK // 16, add_off, 0, unroll=4)

    def fire_rows(blk):
        p = blk % 2
        cu = pltpu.async_copy(my_scr.at[idx_u.at[p]], rows_u.at[p], sem_u)
        ci = pltpu.async_copy(my_scr.at[idx_i.at[p]], rows_i.at[p], sem_i)
        return cu, ci

    i0 = fire_idx(0)
    i1 = fire_idx(1)
    i0[0].wait()
    i0[1].wait()
    adjust_items(0)
    g_pend = fire_rows(0)
    o_pend = None
    idx_pend = i1

    for blk in range(nblk):
        p = blk % 2
        g_pend[0].wait()
        g_pend[1].wait()
        if blk + 1 < nblk:
            idx_pend[0].wait()
            idx_pend[1].wait()
            adjust_items((blk + 1) % 2)
            if blk + 2 < nblk:
                idx_pend = fire_idx(blk + 2)
            g_pend = fire_rows(blk + 1)
        if o_pend is not None:
            o_pend.wait()

        ru = rows_u.at[p]
        ri = rows_i.at[p]

        def group2(g, carry, _ru=ru, _ri=ri, _p=p):
            rid = g * 16 + lane
            acc = None
            for k in range(EMB):
                uk = plsc.load_gather(_ru, [rid, cols[k]])
                ik = plsc.load_gather(_ri, [rid, cols[k]])
                prod = uk * ik
                acc = prod if acc is None else acc + prod
            out_blk[_p, pl.ds(g * 16, 16)] = acc
            return carry

        lax.fori_loop(0, BLOCK // 16, group2, 0, unroll=2)

        off = base + blk * BLOCK
        o_pend = pltpu.async_copy(out_blk.at[p],
                                  out_hbm.at[pl.ds(off, BLOCK)], sem_o)
    o_pend.wait()


def kernel(user, item, users, items):
    B, L = user.shape
    n = B * L
    per_w = n // NUM_WORKERS
    nblk = per_w // BLOCK
    nrows = users.shape[0]
    assert per_w * NUM_WORKERS == n and nblk * BLOCK == per_w
    assert nrows == items.shape[0] and (2 * nrows) % 128 == 0

    combined = jnp.concatenate([users, items], axis=0)
    x = combined.T.reshape(2, 8, NCB, 128).transpose(0, 2, 1, 3)

    mesh = plsc.VectorSubcoreMesh(core_axis_name="c", subcore_axis_name="s")
    f = pl.kernel(
        functools.partial(_mf_body, nblk, nrows),
        out_type=jax.ShapeDtypeStruct((n,), jnp.float32),
        mesh=mesh,
        scratch_types=[
            pltpu.HBM((2, 2 * nrows, EMB), jnp.float32),
            pltpu.VMEM((2, 2, CCB, 8, 128), jnp.float32),
            pltpu.VMEM((2, CP, EMB), jnp.float32),
            pltpu.VMEM((2, BLOCK), jnp.int32),
            pltpu.VMEM((2, BLOCK), jnp.int32),
            pltpu.VMEM((2, BLOCK, EMB), jnp.float32),
            pltpu.VMEM((2, BLOCK, EMB), jnp.float32),
            pltpu.VMEM((2, BLOCK), jnp.float32),
            pltpu.SemaphoreType.DMA,
            pltpu.SemaphoreType.DMA,
            pltpu.SemaphoreType.DMA,
            pltpu.SemaphoreType.DMA,
            pltpu.SemaphoreType.DMA,
            pltpu.SemaphoreType.DMA,
            pltpu.SemaphoreType.DMA,
        ],
        compiler_params=pltpu.CompilerParams(
            use_tc_tiling_on_sc=False, needs_layout_passes=False),
    )
    out = f(user.reshape(-1).astype(jnp.int32),
            item.reshape(-1).astype(jnp.int32),
            x)
    return out.reshape(B, L)


# speedup vs baseline: 1.6910x; 1.1894x over previous
import functools

import jax
import jax.numpy as jnp
from jax import lax
from jax.experimental import pallas as pl
from jax.experimental.pallas import tpu as pltpu
from jax.experimental.pallas import tpu_sc as plsc

EMB = 16
KH = 8
BLOCK = 1024
NCB = 15625
CCB = 5
CP = CCB * 128


def _mf_body(nblk, nrows, uidx_hbm, iidx_hbm, x_hbm, out_hbm,
             scr, tin, rm, idx_u, idx_i, rows_u, rows_i, out_blk,
             sem_t, sem_r, sem_iu, sem_ii, sem_u, sem_i, sem_o):
    cid = lax.axis_index("c")
    sid = lax.axis_index("s")
    lane = lax.iota(jnp.int32, 16)
    cols = [jnp.full((16,), k, jnp.int32) for k in range(KH)]

    nch_all = NCB // CCB
    base_ch = nch_all // 16
    rem_ch = nch_all - base_ch * 16
    nch = jnp.where(sid < rem_ch, base_ch + 1, base_ch)
    ch0 = sid * base_ch + jnp.minimum(sid, rem_ch)

    def fire_in(ch, p):
        cb = (ch0 + ch) * CCB
        return pltpu.async_copy(x_hbm.at[cid, pl.ds(cb, CCB)], tin.at[p],
                                sem_t)

    def transpose_chunk(p):
        for cbl in range(CCB):
            def grp(c8, rowv, _cbl=cbl, _p=p):
                cc = c8 * 16
                for k in range(KH):
                    v = tin[_p, _cbl, k, pl.ds(cc, 16)]
                    plsc.store_scatter(rm.at[_p], [rowv, cols[k]], v)
                return rowv + 16
            lax.fori_loop(0, 8, grp, cbl * 128 + lane, unroll=4)

    def chunk_body(ch, carry):
        p = ch % 2
        pltpu.make_async_copy(x_hbm.at[0, pl.ds(0, CCB)], tin.at[p],
                              sem_t).wait()

        @pl.when(ch >= 2)
        def _():
            pltpu.make_async_copy(rm.at[p], scr.at[0, pl.ds(0, CP)],
                                  sem_r).wait()

        @pl.when(p == 0)
        def _():
            transpose_chunk(0)

        @pl.when(p == 1)
        def _():
            transpose_chunk(1)

        @pl.when(ch + 2 < nch)
        def _():
            fire_in(ch + 2, p)

        p0 = (ch0 + ch) * CP
        pltpu.async_copy(rm.at[p], scr.at[cid, pl.ds(p0, CP)], sem_r)
        return carry

    fire_in(0, 0)

    @pl.when(nch > 1)
    def _():
        fire_in(1, 1)

    lax.fori_loop(0, nch, chunk_body, 0)
    pltpu.make_async_copy(rm.at[0], scr.at[0, pl.ds(0, CP)], sem_r).wait()

    @pl.when(nch > 1)
    def _():
        pltpu.make_async_copy(rm.at[1], scr.at[0, pl.ds(0, CP)],
                              sem_r).wait()

    plsc.subcore_barrier()

    base = sid * (nblk * BLOCK)

    def fire_idx(blk):
        p = blk % 2
        off = base + blk * BLOCK
        cu = pltpu.async_copy(uidx_hbm.at[pl.ds(off, BLOCK)], idx_u.at[p],
                              sem_iu)
        ci = pltpu.async_copy(iidx_hbm.at[pl.ds(off, BLOCK)], idx_i.at[p],
                              sem_ii)
        return cu, ci

    def adjust_items(p):
        def add_off(g, _):
            s = g * 16
            idx_i[p, pl.ds(s, 16)] = idx_i[p, pl.ds(s, 16)] + nrows
            return _
        lax.fori_loop(0, BLOCK // 16, add_off, 0, unroll=4)

    my_scr = scr.at[cid]

    def fire_rows(blk):
        p = blk % 2
        cu = pltpu.async_copy(my_scr.at[idx_u.at[p]], rows_u.at[p], sem_u)
        ci = pltpu.async_copy(my_scr.at[idx_i.at[p]], rows_i.at[p], sem_i)
        return cu, ci

    i0 = fire_idx(0)
    i1 = fire_idx(1)
    i0[0].wait()
    i0[1].wait()
    adjust_items(0)
    g_pend = fire_rows(0)
    o_pend = None
    idx_pend = i1

    for blk in range(nblk):
        p = blk % 2
        g_pend[0].wait()
        g_pend[1].wait()
        if blk + 1 < nblk:
            idx_pend[0].wait()
            idx_pend[1].wait()
            adjust_items((blk + 1) % 2)
            if blk + 2 < nblk:
                idx_pend = fire_idx(blk + 2)
            g_pend = fire_rows(blk + 1)
        if o_pend is not None:
            o_pend.wait()

        ru = rows_u.at[p]
        ri = rows_i.at[p]

        def group2b(g, _, _ru=ru, _ri=ri, _p=p):
            rid = g * 16 + lane
            acc = None
            for k in range(KH):
                uk = plsc.load_gather(_ru, [rid, cols[k]])
                ik = plsc.load_gather(_ri, [rid, cols[k]])
                prod = uk * ik
                acc = prod if acc is None else acc + prod
            out_blk[_p, pl.ds(g * 16, 16)] = acc
            return _

        lax.fori_loop(0, BLOCK // 16, group2b, 0, unroll=2)

        off = base + blk * BLOCK
        o_pend = pltpu.async_copy(out_blk.at[p],
                                  out_hbm.at[cid, pl.ds(off, BLOCK)], sem_o)
    o_pend.wait()


def kernel(user, item, users, items):
    B, L = user.shape
    n = B * L
    per_w = n // 16
    nblk = per_w // BLOCK
    nrows = users.shape[0]
    assert per_w * 16 == n and nblk * BLOCK == per_w
    assert nrows == items.shape[0] and (2 * nrows) % 128 == 0

    combined = jnp.concatenate([users, items], axis=0)
    x = combined.T.reshape(2, 8, NCB, 128).transpose(0, 2, 1, 3)

    mesh = plsc.VectorSubcoreMesh(core_axis_name="c", subcore_axis_name="s")
    f = pl.kernel(
        functools.partial(_mf_body, nblk, nrows),
        out_type=jax.ShapeDtypeStruct((2, n), jnp.float32),
        mesh=mesh,
        scratch_types=[
            pltpu.HBM((2, 2 * nrows, KH), jnp.float32),
            pltpu.VMEM((2, CCB, 8, 128), jnp.float32),
            pltpu.VMEM((2, CP, KH), jnp.float32),
            pltpu.VMEM((2, BLOCK), jnp.int32),
            pltpu.VMEM((2, BLOCK), jnp.int32),
            pltpu.VMEM((2, BLOCK, KH), jnp.float32),
            pltpu.VMEM((2, BLOCK, KH), jnp.float32),
            pltpu.VMEM((2, BLOCK), jnp.float32),
            pltpu.SemaphoreType.DMA,
            pltpu.SemaphoreType.DMA,
            pltpu.SemaphoreType.DMA,
            pltpu.SemaphoreType.DMA,
            pltpu.SemaphoreType.DMA,
            pltpu.SemaphoreType.DMA,
            pltpu.SemaphoreType.DMA,
        ],
        compiler_params=pltpu.CompilerParams(
            use_tc_tiling_on_sc=False, needs_layout_passes=False),
    )
    parts = f(user.reshape(-1).astype(jnp.int32),
              item.reshape(-1).astype(jnp.int32),
              x)
    out = parts[0] + parts[1]
    return out.reshape(B, L)


# speedup vs baseline: 2.0910x; 1.2365x over previous
import functools

import jax
import jax.numpy as jnp
from jax import lax
from jax.experimental import pallas as pl
from jax.experimental.pallas import tpu as pltpu
from jax.experimental.pallas import tpu_sc as plsc

EMB = 16
KH = 8
BLOCK = 1024
NCB = 15625
CCB = 5
CP = CCB * 128


def _mf_body(nblk, nrows, uidx_hbm, iidx_hbm, x_hbm, out_hbm,
             scr, tin, rm, idx_u, idx_i, rows_u, rows_i, out_blk,
             sem_t, sem_r, sem_iu, sem_ii, sem_u, sem_i, sem_o):
    cid = lax.axis_index("c")
    sid = lax.axis_index("s")
    lane = lax.iota(jnp.int32, 16)
    cols = [jnp.full((16,), k, jnp.int32) for k in range(KH)]

    nch_all = NCB // CCB
    base_ch = nch_all // 16
    rem_ch = nch_all - base_ch * 16
    nch = jnp.where(sid < rem_ch, base_ch + 1, base_ch)
    ch0 = sid * base_ch + jnp.minimum(sid, rem_ch)

    def fire_in(ch, p):
        cb = (ch0 + ch) * CCB
        return pltpu.async_copy(x_hbm.at[cid, pl.ds(cb, CCB)], tin.at[p],
                                sem_t)

    def transpose_chunk(p):
        for cbl in range(CCB):
            def grp(c8, rowv, _cbl=cbl, _p=p):
                cc = c8 * 16
                for k in range(KH):
                    v = tin[_p, _cbl, k, pl.ds(cc, 16)]
                    plsc.store_scatter(rm.at[_p], [rowv, cols[k]], v)
                return rowv + 16
            lax.fori_loop(0, 8, grp, cbl * 128 + lane, unroll=8)

    def chunk_body(ch, carry):
        p = ch % 2
        pltpu.make_async_copy(x_hbm.at[0, pl.ds(0, CCB)], tin.at[p],
                              sem_t).wait()

        @pl.when(ch >= 2)
        def _():
            pltpu.make_async_copy(rm.at[p], scr.at[0, pl.ds(0, CP)],
                                  sem_r).wait()

        @pl.when(p == 0)
        def _():
            transpose_chunk(0)

        @pl.when(p == 1)
        def _():
            transpose_chunk(1)

        @pl.when(ch + 2 < nch)
        def _():
            fire_in(ch + 2, p)

        p0 = (ch0 + ch) * CP
        pltpu.async_copy(rm.at[p], scr.at[cid, pl.ds(p0, CP)], sem_r)
        return carry

    fire_in(0, 0)

    @pl.when(nch > 1)
    def _():
        fire_in(1, 1)

    lax.fori_loop(0, nch, chunk_body, 0)
    pltpu.make_async_copy(rm.at[0], scr.at[0, pl.ds(0, CP)], sem_r).wait()

    @pl.when(nch > 1)
    def _():
        pltpu.make_async_copy(rm.at[1], scr.at[0, pl.ds(0, CP)],
                              sem_r).wait()

    plsc.subcore_barrier()

    base = sid * (nblk * BLOCK)

    def fire_idx(blk):
        p = blk % 2
        off = base + blk * BLOCK
        cu = pltpu.async_copy(uidx_hbm.at[pl.ds(off, BLOCK)], idx_u.at[p],
                              sem_iu)
        ci = pltpu.async_copy(iidx_hbm.at[pl.ds(off, BLOCK)], idx_i.at[p],
                              sem_ii)
        return cu, ci

    def adjust_items(p):
        def add_off(g, _):
            s = g * 16
            idx_i[p, pl.ds(s, 16)] = idx_i[p, pl.ds(s, 16)] + nrows
            return _
        lax.fori_loop(0, BLOCK // 16, add_off, 0, unroll=4)

    my_scr = scr.at[cid]

    def fire_rows(blk):
        p = blk % 2
        cu = pltpu.async_copy(my_scr.at[idx_u.at[p]], rows_u.at[p], sem_u)
        ci = pltpu.async_copy(my_scr.at[idx_i.at[p]], rows_i.at[p], sem_i)
        return cu, ci

    i0 = fire_idx(0)
    i1 = fire_idx(1)
    i0[0].wait()
    i0[1].wait()
    adjust_items(0)
    g_pend = fire_rows(0)
    o_pend = None
    idx_pend = i1

    for blk in range(nblk):
        p = blk % 2
        g_pend[0].wait()
        g_pend[1].wait()
        if blk + 1 < nblk:
            idx_pend[0].wait()
            idx_pend[1].wait()
            adjust_items((blk + 1) % 2)
            if blk + 2 < nblk:
                idx_pend = fire_idx(blk + 2)
            g_pend = fire_rows(blk + 1)
        if o_pend is not None:
            o_pend.wait()

        ru = rows_u.at[p]
        ri = rows_i.at[p]

        def group2b(g, _, _ru=ru, _ri=ri, _p=p):
            rid = g * 16 + lane
            acc = None
            for k in range(KH):
                uk = plsc.load_gather(_ru, [rid, cols[k]])
                ik = plsc.load_gather(_ri, [rid, cols[k]])
                prod = uk * ik
                acc = prod if acc is None else acc + prod
            out_blk[_p, pl.ds(g * 16, 16)] = acc
            return _

        lax.fori_loop(0, BLOCK // 16, group2b, 0, unroll=4)

        off = base + blk * BLOCK
        o_pend = pltpu.async_copy(out_blk.at[p],
                                  out_hbm.at[cid, pl.ds(off, BLOCK)], sem_o)
    o_pend.wait()


def kernel(user, item, users, items):
    B, L = user.shape
    n = B * L
    per_w = n // 16
    nblk = per_w // BLOCK
    nrows = users.shape[0]
    assert per_w * 16 == n and nblk * BLOCK == per_w
    assert nrows == items.shape[0] and (2 * nrows) % 128 == 0

    combined = jnp.concatenate([users, items], axis=0)
    x = combined.T.reshape(2, 8, NCB, 128).transpose(0, 2, 1, 3)

    mesh = plsc.VectorSubcoreMesh(core_axis_name="c", subcore_axis_name="s")
    f = pl.kernel(
        functools.partial(_mf_body, nblk, nrows),
        out_type=jax.ShapeDtypeStruct((2, n), jnp.float32),
        mesh=mesh,
        scratch_types=[
            pltpu.HBM((2, 2 * nrows, KH), jnp.float32),
            pltpu.VMEM((2, CCB, 8, 128), jnp.float32),
            pltpu.VMEM((2, CP, KH), jnp.float32),
            pltpu.VMEM((2, BLOCK), jnp.int32),
            pltpu.VMEM((2, BLOCK), jnp.int32),
            pltpu.VMEM((2, BLOCK, KH), jnp.float32),
            pltpu.VMEM((2, BLOCK, KH), jnp.float32),
            pltpu.VMEM((2, BLOCK), jnp.float32),
            pltpu.SemaphoreType.DMA,
            pltpu.SemaphoreType.DMA,
            pltpu.SemaphoreType.DMA,
            pltpu.SemaphoreType.DMA,
            pltpu.SemaphoreType.DMA,
            pltpu.SemaphoreType.DMA,
            pltpu.SemaphoreType.DMA,
        ],
        compiler_params=pltpu.CompilerParams(
            use_tc_tiling_on_sc=False, needs_layout_passes=False),
    )
    parts = f(user.reshape(-1).astype(jnp.int32),
              item.reshape(-1).astype(jnp.int32),
              x)
    out = parts[0] + parts[1]
    return out.reshape(B, L)


# speedup vs baseline: 2.1010x; 1.0048x over previous
import functools

import jax
import jax.numpy as jnp
from jax import lax
from jax.experimental import pallas as pl
from jax.experimental.pallas import tpu as pltpu
from jax.experimental.pallas import tpu_sc as plsc

EMB = 16
KH = 8
BLOCK = 1024
NCB = 15625
CCB = 5
CP = CCB * 128


def _mf_body(nblk, nrows, uidx_hbm, iidx_hbm, x_hbm, out_hbm,
             scr, tin, rm, idx_u, idx_i, rows_u, rows_i, out_blk,
             sem_t, sem_r, sem_iu, sem_ii, sem_u, sem_i, sem_o):
    cid = lax.axis_index("c")
    sid = lax.axis_index("s")
    lane = lax.iota(jnp.int32, 16)
    cols = [jnp.full((16,), k, jnp.int32) for k in range(KH)]

    nch_all = NCB // CCB
    base_ch = nch_all // 16
    rem_ch = nch_all - base_ch * 16
    nch = jnp.where(sid < rem_ch, base_ch + 1, base_ch)
    ch0 = sid * base_ch + jnp.minimum(sid, rem_ch)

    def fire_in(ch, p):
        cb = (ch0 + ch) * CCB
        return pltpu.async_copy(x_hbm.at[cid, pl.ds(cb, CCB)], tin.at[p],
                                sem_t)

    def transpose_chunk(p):
        for cbl in range(CCB):
            def grp(c8, rowv, _cbl=cbl, _p=p):
                cc = c8 * 16
                for k in range(KH):
                    v = tin[_p, _cbl, k, pl.ds(cc, 16)]
                    plsc.store_scatter(rm.at[_p], [rowv, cols[k]], v)
                return rowv + 16
            lax.fori_loop(0, 8, grp, cbl * 128 + lane, unroll=8)

    def chunk_body(ch, carry):
        p = ch % 2
        pltpu.make_async_copy(x_hbm.at[0, pl.ds(0, CCB)], tin.at[p],
                              sem_t).wait()

        @pl.when(ch >= 2)
        def _():
            pltpu.make_async_copy(rm.at[p], scr.at[0, pl.ds(0, CP)],
                                  sem_r).wait()

        @pl.when(p == 0)
        def _():
            transpose_chunk(0)

        @pl.when(p == 1)
        def _():
            transpose_chunk(1)

        @pl.when(ch + 2 < nch)
        def _():
            fire_in(ch + 2, p)

        p0 = (ch0 + ch) * CP
        pltpu.async_copy(rm.at[p], scr.at[cid, pl.ds(p0, CP)], sem_r)
        return carry

    fire_in(0, 0)

    @pl.when(nch > 1)
    def _():
        fire_in(1, 1)

    lax.fori_loop(0, nch, chunk_body, 0)
    pltpu.make_async_copy(rm.at[0], scr.at[0, pl.ds(0, CP)], sem_r).wait()

    @pl.when(nch > 1)
    def _():
        pltpu.make_async_copy(rm.at[1], scr.at[0, pl.ds(0, CP)],
                              sem_r).wait()

    plsc.subcore_barrier()

    base = sid * (nblk * BLOCK)

    def fire_idx(blk):
        p = blk % 2
        off = base + blk * BLOCK
        cu = pltpu.async_copy(uidx_hbm.at[pl.ds(off, BLOCK)], idx_u.at[p],
                              sem_iu)
        ci = pltpu.async_copy(iidx_hbm.at[pl.ds(off, BLOCK)], idx_i.at[p],
                              sem_ii)
        return cu, ci

    def adjust_items(p):
        def add_off(g, _):
            s = g * 16
            idx_i[p, pl.ds(s, 16)] = idx_i[p, pl.ds(s, 16)] + nrows
            return _
        lax.fori_loop(0, BLOCK // 16, add_off, 0, unroll=4)

    my_scr = scr.at[cid]

    def fire_rows(blk):
        p = blk % 2
        cu = pltpu.async_copy(my_scr.at[idx_u.at[p]], rows_u.at[p], sem_u)
        ci = pltpu.async_copy(my_scr.at[idx_i.at[p]], rows_i.at[p], sem_i)
        return cu, ci

    def wait_idx():
        pltpu.make_async_copy(uidx_hbm.at[pl.ds(0, BLOCK)], idx_u.at[0],
                              sem_iu).wait()
        pltpu.make_async_copy(iidx_hbm.at[pl.ds(0, BLOCK)], idx_i.at[0],
                              sem_ii).wait()

    def wait_rows():
        pltpu.make_async_copy(my_scr.at[idx_u.at[0]], rows_u.at[0],
                              sem_u).wait()
        pltpu.make_async_copy(my_scr.at[idx_i.at[0]], rows_i.at[0],
                              sem_i).wait()

    def wait_out():
        pltpu.make_async_copy(out_blk.at[0],
                              out_hbm.at[0, pl.ds(0, BLOCK)], sem_o).wait()

    def dot_block(p):
        def group2(g, _, _p=p):
            rid = g * 16 + lane
            acc = None
            for k in range(KH):
                uk = plsc.load_gather(rows_u.at[_p], [rid, cols[k]])
                ik = plsc.load_gather(rows_i.at[_p], [rid, cols[k]])
                prod = uk * ik
                acc = prod if acc is None else acc + prod
            out_blk[_p, pl.ds(g * 16, 16)] = acc
            return _
        lax.fori_loop(0, BLOCK // 16, group2, 0, unroll=8)

    fire_idx(0)
    fire_idx(1)
    wait_idx()
    adjust_items(0)
    fire_rows(0)

    def blk_body(blk, carry):
        p = blk % 2
        wait_rows()

        @pl.when(blk + 1 < nblk)
        def _():
            wait_idx()
            nxt = blk + 1

            @pl.when(nxt % 2 == 0)
            def _():
                adjust_items(0)

            @pl.when(nxt % 2 == 1)
            def _():
                adjust_items(1)

            @pl.when(blk + 2 < nblk)
            def _():
                fire_idx(blk + 2)

            fire_rows(blk + 1)

        @pl.when(blk >= 2)
        def _():
            wait_out()

        @pl.when(p == 0)
        def _():
            dot_block(0)

        @pl.when(p == 1)
        def _():
            dot_block(1)

        off = base + blk * BLOCK
        pltpu.async_copy(out_blk.at[p], out_hbm.at[cid, pl.ds(off, BLOCK)],
                         sem_o)
        return carry

    lax.fori_loop(0, nblk, blk_body, 0)
    wait_out()

    @pl.when(nblk > 1)
    def _():
        wait_out()


def kernel(user, item, users, items):
    B, L = user.shape
    n = B * L
    per_w = n // 16
    nblk = per_w // BLOCK
    nrows = users.shape[0]
    assert per_w * 16 == n and nblk * BLOCK == per_w
    assert nrows == items.shape[0] and (2 * nrows) % 128 == 0

    combined = jnp.concatenate([users, items], axis=0)
    x = combined.T.reshape(2, 8, NCB, 128).transpose(0, 2, 1, 3)

    mesh = plsc.VectorSubcoreMesh(core_axis_name="c", subcore_axis_name="s")
    f = pl.kernel(
        functools.partial(_mf_body, nblk, nrows),
        out_type=jax.ShapeDtypeStruct((2, n), jnp.float32),
        mesh=mesh,
        scratch_types=[
            pltpu.HBM((2, 2 * nrows, KH), jnp.float32),
            pltpu.VMEM((2, CCB, 8, 128), jnp.float32),
            pltpu.VMEM((2, CP, KH), jnp.float32),
            pltpu.VMEM((2, BLOCK), jnp.int32),
            pltpu.VMEM((2, BLOCK), jnp.int32),
            pltpu.VMEM((2, BLOCK, KH), jnp.float32),
            pltpu.VMEM((2, BLOCK, KH), jnp.float32),
            pltpu.VMEM((2, BLOCK), jnp.float32),
            pltpu.SemaphoreType.DMA,
            pltpu.SemaphoreType.DMA,
            pltpu.SemaphoreType.DMA,
            pltpu.SemaphoreType.DMA,
            pltpu.SemaphoreType.DMA,
            pltpu.SemaphoreType.DMA,
            pltpu.SemaphoreType.DMA,
        ],
        compiler_params=pltpu.CompilerParams(
            use_tc_tiling_on_sc=False, needs_layout_passes=False),
    )
    parts = f(user.reshape(-1).astype(jnp.int32),
              item.reshape(-1).astype(jnp.int32),
              x)
    out = parts[0] + parts[1]
    return out.reshape(B, L)
